# Initial kernel scaffold; baseline (speedup 1.0000x reference)
import jax
import jax.numpy as jnp
from jax.experimental import pallas as pl


def kernel(x, src0, dst0, src1, dst1, W1, b1, W2, b2):
    raise NotImplementedError("write your pallas kernel here")


# speedup vs baseline: 1.8348x; 1.8348x over previous
import functools

import jax
import jax.numpy as jnp
from jax import lax
from jax.experimental import pallas as pl
from jax.experimental.pallas import tpu as pltpu
from jax.experimental.pallas import tpu_sc as plsc

_N1, _N2 = 4000, 1000
_D_IN, _D_HID, _D_OUT = 256, 512, 256

_NC, _NS = 2, 16
_NW = _NC * _NS
_K = 128
_G = 128


def _make_agg(R, D, CH):
    G = D // _G
    stripe = R // _NS
    mesh = plsc.VectorSubcoreMesh(core_axis_name="c", subcore_axis_name="s")

    @functools.partial(
        pl.kernel,
        mesh=mesh,
        out_type=jax.ShapeDtypeStruct((_NC, G, R, _G), jnp.float32),
        scratch_types=[
            pltpu.VMEM((_K,), jnp.int32),
            pltpu.VMEM((_K,), jnp.int32),
            pltpu.VMEM((_K,), jnp.int32),
            pltpu.VMEM((_K, _G), jnp.float32),
            pltpu.VMEM_SHARED((G, R, _G), jnp.float32),
            pltpu.SemaphoreType.DMA,
        ],
    )
    def agg(feat_hbm, src_hbm, dst_hbm, zrow_hbm,
            out_hbm, didx, sidx, gidx, rows, acc, sem):
        cid = lax.axis_index("c")
        sid = lax.axis_index("s")
        w = cid * _NS + sid
        row0 = sid * stripe
        for g in range(G):
            pltpu.sync_copy(zrow_hbm.at[pl.ds(row0, stripe)],
                            acc.at[g, pl.ds(row0, stripe)])
        plsc.subcore_barrier()

        def body(c, carry):
            base = (w * CH + c) * _K
            pltpu.sync_copy(src_hbm.at[pl.ds(base, _K)], sidx)
            pltpu.sync_copy(dst_hbm.at[pl.ds(base, _K)], didx)
            for g in range(G):
                for j in range(_K // 16):
                    sl = pl.ds(j * 16, 16)
                    gidx[sl] = sidx[sl] * G + g
                pltpu.async_copy(feat_hbm.at[gidx], rows, sem).wait()
                pltpu.sync_copy(rows, acc.at[g].at[didx], add=True)
            return carry

        lax.fori_loop(0, CH, body, 0)
        plsc.subcore_barrier()
        for g in range(G):
            pltpu.sync_copy(acc.at[g, pl.ds(row0, stripe)],
                            out_hbm.at[cid, g, pl.ds(row0, stripe)])

    return agg


def _make_counts(R1, CH1, R2, CH2):
    s1, s2 = R1 // _NS, R2 // _NS
    mesh = plsc.VectorSubcoreMesh(core_axis_name="c", subcore_axis_name="s")

    @functools.partial(
        pl.kernel,
        mesh=mesh,
        out_type=(
            jax.ShapeDtypeStruct((_NC, R1, _G), jnp.float32),
            jax.ShapeDtypeStruct((_NC, R2, _G), jnp.float32),
        ),
        scratch_types=[
            pltpu.VMEM((_K,), jnp.int32),
            pltpu.VMEM((_K, _G), jnp.float32),
            pltpu.VMEM_SHARED((R1, _G), jnp.float32),
            pltpu.VMEM_SHARED((R2, _G), jnp.float32),
        ],
    )
    def cntk(dst1_hbm, dst2_hbm, zrow_hbm, ones_hbm,
             cnt1_hbm, cnt2_hbm, didx, ones, acc1, acc2):
        cid = lax.axis_index("c")
        sid = lax.axis_index("s")
        w = cid * _NS + sid
        pltpu.sync_copy(zrow_hbm.at[pl.ds(sid * s1, s1)],
                        acc1.at[pl.ds(sid * s1, s1)])
        pltpu.sync_copy(zrow_hbm.at[pl.ds(sid * s2, s2)],
                        acc2.at[pl.ds(sid * s2, s2)])
        pltpu.sync_copy(ones_hbm, ones)
        plsc.subcore_barrier()

        def body1(c, carry):
            pltpu.sync_copy(dst1_hbm.at[pl.ds((w * CH1 + c) * _K, _K)], didx)
            pltpu.sync_copy(ones, acc1.at[didx], add=True)
            return carry

        def body2(c, carry):
            pltpu.sync_copy(dst2_hbm.at[pl.ds((w * CH2 + c) * _K, _K)], didx)
            pltpu.sync_copy(ones, acc2.at[didx], add=True)
            return carry

        lax.fori_loop(0, CH1, body1, 0)
        lax.fori_loop(0, CH2, body2, 0)
        plsc.subcore_barrier()
        pltpu.sync_copy(acc1.at[pl.ds(sid * s1, s1)],
                        cnt1_hbm.at[cid, pl.ds(sid * s1, s1)])
        pltpu.sync_copy(acc2.at[pl.ds(sid * s2, s2)],
                        cnt2_hbm.at[cid, pl.ds(sid * s2, s2)])

    return cntk


def _mean_linear_relu(parts, cnts, W, b, bm):
    G, R = parts.shape[1], parts.shape[2]
    Dout = W.shape[1]

    def body(*refs):
        p_refs = refs[: _NC * G]
        c_refs = refs[_NC * G: _NC * G + _NC]
        w_ref, b_ref, o_ref = refs[_NC * G + _NC:]
        cnt = sum(c[:, 0:1] for c in c_refs)
        inv = 1.0 / jnp.maximum(cnt, 1.0)
        acc = jnp.zeros(o_ref.shape, jnp.float32)
        for g in range(G):
            p = p_refs[g][...]
            for c in range(1, _NC):
                p = p + p_refs[c * G + g][...]
            h = p * inv
            acc = acc + jnp.dot(h, w_ref[pl.ds(g * _G, _G), :],
                                preferred_element_type=jnp.float32)
        o_ref[...] = jax.nn.relu(acc + b_ref[...])

    args = ([parts[c, g] for c in range(_NC) for g in range(G)]
            + [cnts[c] for c in range(_NC)] + [W, b.reshape(1, Dout)])
    in_specs = ([pl.BlockSpec((bm, _G), lambda i: (i, 0))] * (_NC * G + _NC)
                + [pl.BlockSpec(W.shape, lambda i: (0, 0)),
                   pl.BlockSpec((1, Dout), lambda i: (0, 0))])
    return pl.pallas_call(
        body,
        grid=(R // bm,),
        in_specs=in_specs,
        out_specs=pl.BlockSpec((bm, Dout), lambda i: (i, 0)),
        out_shape=jax.ShapeDtypeStruct((R, Dout), jnp.float32),
    )(*args)


def _pad_edges(src, dst, e_pad, dummy_dst):
    pad = e_pad - src.shape[0]
    s = jnp.concatenate([src.astype(jnp.int32), jnp.zeros((pad,), jnp.int32)])
    d = jnp.concatenate([dst.astype(jnp.int32),
                         jnp.full((pad,), dummy_dst, jnp.int32)])
    return s, d, e_pad // (_NW * _K)


def kernel(x, src0, dst0, src1, dst1, W1, b1, W2, b2):
    R1, R2 = 4096, 1024
    ones = jnp.ones((_K, _G), jnp.float32)

    s0, d0, ch0 = _pad_edges(src0, dst0, 65536, _N1)
    s1, d1, ch1 = _pad_edges(src1, dst1, 16384, _N2)
    zrow = jnp.zeros((R1, _G), jnp.float32)

    C1, C2 = _make_counts(R1, ch0, R2, ch1)(d0, d1, zrow, ones)

    P1 = _make_agg(R1, _D_IN, ch0)(x.reshape(-1, _G), s0, d0, zrow)
    h1 = _mean_linear_relu(P1, C1, W1, b1, bm=256)

    P2 = _make_agg(R2, _D_HID, ch1)(h1.reshape(-1, _G), s1, d1, zrow)
    h2 = _mean_linear_relu(P2, C2, W2, b2, bm=256)
    return h2[:_N2]


# speedup vs baseline: 3.4923x; 1.9034x over previous
import functools

import jax
import jax.numpy as jnp
from jax import lax
from jax.experimental import pallas as pl
from jax.experimental.pallas import tpu as pltpu
from jax.experimental.pallas import tpu_sc as plsc

_N1, _N2 = 4000, 1000
_D_IN, _D_HID, _D_OUT = 256, 512, 256

_NC, _NS = 2, 16
_NW = _NC * _NS
_K = 128
_G = 128


def _make_agg(R, D, CH):
    G = D // _G
    stripe = R // _NS
    mesh = plsc.VectorSubcoreMesh(core_axis_name="c", subcore_axis_name="s")

    @functools.partial(
        pl.kernel,
        mesh=mesh,
        out_type=jax.ShapeDtypeStruct((_NC, G, R, _G), jnp.float32),
        scratch_types=[
            pltpu.VMEM((_K,), jnp.int32),
            pltpu.VMEM((_K,), jnp.int32),
            pltpu.VMEM((_K,), jnp.int32),
            pltpu.VMEM((_K, _G), jnp.float32),
            pltpu.VMEM_SHARED((G, R, _G), jnp.float32),
            pltpu.SemaphoreType.DMA,
        ],
    )
    def agg(feat_hbm, src_hbm, dst_hbm, zrow_hbm,
            out_hbm, didx, sidx, gidx, rows, acc, sem):
        cid = lax.axis_index("c")
        sid = lax.axis_index("s")
        w = cid * _NS + sid
        row0 = sid * stripe
        for g in range(G):
            pltpu.sync_copy(zrow_hbm.at[pl.ds(row0, stripe)],
                            acc.at[g, pl.ds(row0, stripe)])
        plsc.subcore_barrier()

        def body(c, carry):
            base = (w * CH + c) * _K
            pltpu.sync_copy(src_hbm.at[pl.ds(base, _K)], sidx)
            pltpu.sync_copy(dst_hbm.at[pl.ds(base, _K)], didx)
            for g in range(G):
                for j in range(_K // 16):
                    sl = pl.ds(j * 16, 16)
                    gidx[sl] = sidx[sl] * G + g
                pltpu.async_copy(feat_hbm.at[gidx], rows, sem).wait()
                pltpu.sync_copy(rows, acc.at[g].at[didx], add=True)
            return carry

        lax.fori_loop(0, CH, body, 0)
        plsc.subcore_barrier()
        for g in range(G):
            pltpu.sync_copy(acc.at[g, pl.ds(row0, stripe)],
                            out_hbm.at[cid, g, pl.ds(row0, stripe)])

    return agg


def _make_counts(R1, CH1, R2, CH2):
    s1, s2 = R1 // _NS, R2 // _NS
    mesh = plsc.VectorSubcoreMesh(core_axis_name="c", subcore_axis_name="s")

    @functools.partial(
        pl.kernel,
        mesh=mesh,
        out_type=(
            jax.ShapeDtypeStruct((_NC, R1, _G), jnp.float32),
            jax.ShapeDtypeStruct((_NC, R2, _G), jnp.float32),
        ),
        scratch_types=[
            pltpu.VMEM((_K,), jnp.int32),
            pltpu.VMEM((_K, _G), jnp.float32),
            pltpu.VMEM_SHARED((R1, _G), jnp.float32),
            pltpu.VMEM_SHARED((R2, _G), jnp.float32),
        ],
    )
    def cntk(dst1_hbm, dst2_hbm, zrow_hbm, ones_hbm,
             cnt1_hbm, cnt2_hbm, didx, ones, acc1, acc2):
        cid = lax.axis_index("c")
        sid = lax.axis_index("s")
        w = cid * _NS + sid
        pltpu.sync_copy(zrow_hbm.at[pl.ds(sid * s1, s1)],
                        acc1.at[pl.ds(sid * s1, s1)])
        pltpu.sync_copy(zrow_hbm.at[pl.ds(sid * s2, s2)],
                        acc2.at[pl.ds(sid * s2, s2)])
        pltpu.sync_copy(ones_hbm, ones)
        plsc.subcore_barrier()

        def body1(c, carry):
            pltpu.sync_copy(dst1_hbm.at[pl.ds((w * CH1 + c) * _K, _K)], didx)
            pltpu.sync_copy(ones, acc1.at[didx], add=True)
            return carry

        def body2(c, carry):
            pltpu.sync_copy(dst2_hbm.at[pl.ds((w * CH2 + c) * _K, _K)], didx)
            pltpu.sync_copy(ones, acc2.at[didx], add=True)
            return carry

        lax.fori_loop(0, CH1, body1, 0)
        lax.fori_loop(0, CH2, body2, 0)
        plsc.subcore_barrier()
        pltpu.sync_copy(acc1.at[pl.ds(sid * s1, s1)],
                        cnt1_hbm.at[cid, pl.ds(sid * s1, s1)])
        pltpu.sync_copy(acc2.at[pl.ds(sid * s2, s2)],
                        cnt2_hbm.at[cid, pl.ds(sid * s2, s2)])

    return cntk


def _mean_linear_relu(parts, cnts, W, b, bm):
    G, R = parts.shape[1], parts.shape[2]
    Dout = W.shape[1]

    def body(*refs):
        p_refs = refs[: _NC * G]
        c_refs = refs[_NC * G: _NC * G + _NC]
        w_ref, b_ref, o_ref = refs[_NC * G + _NC:]
        cnt = sum(c[:, 0:1] for c in c_refs)
        inv = 1.0 / jnp.maximum(cnt, 1.0)
        acc = jnp.zeros(o_ref.shape, jnp.float32)
        for g in range(G):
            p = p_refs[g][...]
            for c in range(1, _NC):
                p = p + p_refs[c * G + g][...]
            h = p * inv
            acc = acc + jnp.dot(h, w_ref[pl.ds(g * _G, _G), :],
                                preferred_element_type=jnp.float32)
        o_ref[...] = jax.nn.relu(acc + b_ref[...])

    args = ([parts[c, g] for c in range(_NC) for g in range(G)]
            + [cnts[c] for c in range(_NC)] + [W, b.reshape(1, Dout)])
    in_specs = ([pl.BlockSpec((bm, _G), lambda i: (i, 0))] * (_NC * G + _NC)
                + [pl.BlockSpec(W.shape, lambda i: (0, 0)),
                   pl.BlockSpec((1, Dout), lambda i: (0, 0))])
    return pl.pallas_call(
        body,
        grid=(R // bm,),
        in_specs=in_specs,
        out_specs=pl.BlockSpec((bm, Dout), lambda i: (i, 0)),
        out_shape=jax.ShapeDtypeStruct((R, Dout), jnp.float32),
    )(*args)


def _pad_edges(src, dst, e_pad, n_src, dummy_lo, dummy_hi):
    pad = e_pad - src.shape[0]
    i = jnp.arange(pad, dtype=jnp.int32)
    s = jnp.concatenate([src.astype(jnp.int32), i % n_src])
    d = jnp.concatenate([dst.astype(jnp.int32),
                         dummy_lo + i % (dummy_hi - dummy_lo)])
    return s, d, e_pad // (_NW * _K)


def kernel(x, src0, dst0, src1, dst1, W1, b1, W2, b2):
    R1, R2 = 4096, 1024
    ones = jnp.ones((_K, _G), jnp.float32)

    s0, d0, ch0 = _pad_edges(src0, dst0, 65536, 10000, _N1, R1)
    s1, d1, ch1 = _pad_edges(src1, dst1, 16384, R1, _N2, R2)
    zrow = jnp.zeros((R1, _G), jnp.float32)

    C1, C2 = _make_counts(R1, ch0, R2, ch1)(d0, d1, zrow, ones)

    P1 = _make_agg(R1, _D_IN, ch0)(x.reshape(-1, _G), s0, d0, zrow)
    h1 = _mean_linear_relu(P1, C1, W1, b1, bm=256)

    P2 = _make_agg(R2, _D_HID, ch1)(h1.reshape(-1, _G), s1, d1, zrow)
    h2 = _mean_linear_relu(P2, C2, W2, b2, bm=256)
    return h2[:_N2]


# speedup vs baseline: 4.1363x; 1.1844x over previous
import functools

import jax
import jax.numpy as jnp
from jax import lax
from jax.experimental import pallas as pl
from jax.experimental.pallas import tpu as pltpu
from jax.experimental.pallas import tpu_sc as plsc

_N1, _N2 = 4000, 1000
_D_IN, _D_HID, _D_OUT = 256, 512, 256

_NC, _NS = 2, 16
_NW = _NC * _NS
_K = 128
_G = 128


def _make_agg(R, D, CH):
    G = D // _G
    stripe = R // _NS
    mesh = plsc.VectorSubcoreMesh(core_axis_name="c", subcore_axis_name="s")

    @functools.partial(
        pl.kernel,
        mesh=mesh,
        out_type=jax.ShapeDtypeStruct((_NC, G, R, _G), jnp.float32),
        scratch_types=[
            pltpu.VMEM((2, _K), jnp.int32),
            pltpu.VMEM((_K,), jnp.int32),
            pltpu.VMEM((2, _K), jnp.int32),
            pltpu.VMEM((2, _K, _G), jnp.float32),
            pltpu.VMEM_SHARED((G, R, _G), jnp.float32),
            pltpu.SemaphoreType.DMA,
            pltpu.SemaphoreType.DMA,
        ],
    )
    def agg(feat_hbm, src_hbm, dst_hbm, zrow_hbm,
            out_hbm, didx, sidx, gidx, rows, acc, sem0, sem1):
        cid = lax.axis_index("c")
        sid = lax.axis_index("s")
        w = cid * _NS + sid
        row0 = sid * stripe
        sems = (sem0, sem1)
        for g in range(G):
            pltpu.sync_copy(zrow_hbm.at[pl.ds(row0, stripe)],
                            acc.at[g, pl.ds(row0, stripe)])
        plsc.subcore_barrier()

        def load_chunk(c, cpar):
            base = (w * CH + c) * _K
            pltpu.sync_copy(src_hbm.at[pl.ds(base, _K)], sidx)
            pltpu.sync_copy(dst_hbm.at[pl.ds(base, _K)], didx.at[cpar])

        def start_gather(g, spar):
            for j in range(_K // 16):
                sl = pl.ds(j * 16, 16)
                gidx[spar, sl] = sidx[sl] * G + g
            pltpu.async_copy(feat_hbm.at[gidx.at[spar]], rows.at[spar],
                             sems[spar])

        def wait_gather(spar):
            pltpu.make_async_copy(zrow_hbm.at[pl.ds(0, _K)], rows.at[spar],
                                  sems[spar]).wait()

        load_chunk(0, 0)
        start_gather(0, 0)

        def body(i, carry):
            for k in range(2 * G):
                cpar, g, spar = k // G, k % G, k % 2
                nk = k + 1
                if nk % G == 0:
                    load_chunk(2 * i + nk // G, (nk // G) % 2)
                start_gather(nk % G, nk % 2)
                wait_gather(spar)
                pltpu.sync_copy(rows.at[spar], acc.at[g].at[didx.at[cpar]],
                                add=True)
            return carry

        lax.fori_loop(0, CH // 2, body, 0)
        wait_gather(0)
        plsc.subcore_barrier()
        for g in range(G):
            pltpu.sync_copy(acc.at[g, pl.ds(row0, stripe)],
                            out_hbm.at[cid, g, pl.ds(row0, stripe)])

    return agg


def _make_counts(R1, CH1, R2, CH2):
    s1, s2 = R1 // _NS, R2 // _NS
    mesh = plsc.VectorSubcoreMesh(core_axis_name="c", subcore_axis_name="s")

    @functools.partial(
        pl.kernel,
        mesh=mesh,
        out_type=(
            jax.ShapeDtypeStruct((_NC, R1, _G), jnp.float32),
            jax.ShapeDtypeStruct((_NC, R2, _G), jnp.float32),
        ),
        scratch_types=[
            pltpu.VMEM((_K,), jnp.int32),
            pltpu.VMEM((_K, _G), jnp.float32),
            pltpu.VMEM_SHARED((R1, _G), jnp.float32),
            pltpu.VMEM_SHARED((R2, _G), jnp.float32),
        ],
    )
    def cntk(dst1_hbm, dst2_hbm, zrow_hbm, ones_hbm,
             cnt1_hbm, cnt2_hbm, didx, ones, acc1, acc2):
        cid = lax.axis_index("c")
        sid = lax.axis_index("s")
        w = cid * _NS + sid
        pltpu.sync_copy(zrow_hbm.at[pl.ds(sid * s1, s1)],
                        acc1.at[pl.ds(sid * s1, s1)])
        pltpu.sync_copy(zrow_hbm.at[pl.ds(sid * s2, s2)],
                        acc2.at[pl.ds(sid * s2, s2)])
        pltpu.sync_copy(ones_hbm, ones)
        plsc.subcore_barrier()

        def body1(c, carry):
            pltpu.sync_copy(dst1_hbm.at[pl.ds((w * CH1 + c) * _K, _K)], didx)
            pltpu.sync_copy(ones, acc1.at[didx], add=True)
            return carry

        def body2(c, carry):
            pltpu.sync_copy(dst2_hbm.at[pl.ds((w * CH2 + c) * _K, _K)], didx)
            pltpu.sync_copy(ones, acc2.at[didx], add=True)
            return carry

        lax.fori_loop(0, CH1, body1, 0)
        lax.fori_loop(0, CH2, body2, 0)
        plsc.subcore_barrier()
        pltpu.sync_copy(acc1.at[pl.ds(sid * s1, s1)],
                        cnt1_hbm.at[cid, pl.ds(sid * s1, s1)])
        pltpu.sync_copy(acc2.at[pl.ds(sid * s2, s2)],
                        cnt2_hbm.at[cid, pl.ds(sid * s2, s2)])

    return cntk


def _mean_linear_relu(parts, cnts, W, b, bm):
    G, R = parts.shape[1], parts.shape[2]
    Dout = W.shape[1]

    def body(*refs):
        p_refs = refs[: _NC * G]
        c_refs = refs[_NC * G: _NC * G + _NC]
        w_ref, b_ref, o_ref = refs[_NC * G + _NC:]
        cnt = sum(c[:, 0:1] for c in c_refs)
        inv = 1.0 / jnp.maximum(cnt, 1.0)
        acc = jnp.zeros(o_ref.shape, jnp.float32)
        for g in range(G):
            p = p_refs[g][...]
            for c in range(1, _NC):
                p = p + p_refs[c * G + g][...]
            h = p * inv
            acc = acc + jnp.dot(h, w_ref[pl.ds(g * _G, _G), :],
                                preferred_element_type=jnp.float32)
        o_ref[...] = jax.nn.relu(acc + b_ref[...])

    args = ([parts[c, g] for c in range(_NC) for g in range(G)]
            + [cnts[c] for c in range(_NC)] + [W, b.reshape(1, Dout)])
    in_specs = ([pl.BlockSpec((bm, _G), lambda i: (i, 0))] * (_NC * G + _NC)
                + [pl.BlockSpec(W.shape, lambda i: (0, 0)),
                   pl.BlockSpec((1, Dout), lambda i: (0, 0))])
    return pl.pallas_call(
        body,
        grid=(R // bm,),
        in_specs=in_specs,
        out_specs=pl.BlockSpec((bm, Dout), lambda i: (i, 0)),
        out_shape=jax.ShapeDtypeStruct((R, Dout), jnp.float32),
    )(*args)


def _pad_edges(src, dst, e_pad, n_src, dummy_lo, dummy_hi):
    pad = e_pad + _K - src.shape[0]
    i = jnp.arange(pad, dtype=jnp.int32)
    s = jnp.concatenate([src.astype(jnp.int32), i % n_src])
    d = jnp.concatenate([dst.astype(jnp.int32),
                         dummy_lo + i % (dummy_hi - dummy_lo)])
    return s, d, e_pad // (_NW * _K)


def kernel(x, src0, dst0, src1, dst1, W1, b1, W2, b2):
    R1, R2 = 4096, 1024
    ones = jnp.ones((_K, _G), jnp.float32)

    s0, d0, ch0 = _pad_edges(src0, dst0, 65536, 10000, _N1, R1)
    s1, d1, ch1 = _pad_edges(src1, dst1, 16384, R1, _N2, R2)
    zrow = jnp.zeros((R1, _G), jnp.float32)

    C1, C2 = _make_counts(R1, ch0, R2, ch1)(d0, d1, zrow, ones)

    P1 = _make_agg(R1, _D_IN, ch0)(x.reshape(-1, _G), s0, d0, zrow)
    h1 = _mean_linear_relu(P1, C1, W1, b1, bm=256)

    P2 = _make_agg(R2, _D_HID, ch1)(h1.reshape(-1, _G), s1, d1, zrow)
    h2 = _mean_linear_relu(P2, C2, W2, b2, bm=256)
    return h2[:_N2]
